# Initial kernel scaffold; baseline (speedup 1.0000x reference)
import jax
import jax.numpy as jnp
from jax.experimental import pallas as pl


def kernel(x, edge_index, private_state, W0, b0, W1, b1, W2, b2, W3, b3, Wp1, bp1, Wp2, bp2, Wph1, bph1, Wph2, bph2, Wvh1, bvh1, Wvh2, bvh2):
    raise NotImplementedError("write your pallas kernel here")


# speedup vs baseline: 16.2532x; 16.2532x over previous
import functools

import jax
import jax.numpy as jnp
from jax import lax
from jax.experimental import pallas as pl
from jax.experimental.pallas import tpu as pltpu
from jax.experimental.pallas import tpu_sc as plsc

NC = 2
NS = 16
NW = NC * NS
LANES = 16
CHUNK = 128

_MM = functools.partial(jnp.dot, preferred_element_type=jnp.float32)
_MMX = functools.partial(jnp.dot, preferred_element_type=jnp.float32,
                         precision=lax.Precision.HIGHEST)


def _round_up(a, b):
    return (a + b - 1) // b * b


def _sc_mesh():
    return plsc.VectorSubcoreMesh(core_axis_name="c", subcore_axis_name="s",
                                  num_cores=NC, num_subcores=NS)


_SC_PARAMS = pltpu.CompilerParams(use_tc_tiling_on_sc=False)


def _sc_degree(dstp, ones_rows, zeros_rows, np_, chunks):
    rpt = np_ // NS

    @functools.partial(
        pl.kernel,
        out_type=jax.ShapeDtypeStruct((NC, np_, LANES), jnp.float32),
        mesh=_sc_mesh(),
        scratch_types=[
            pltpu.VMEM((chunks, CHUNK), jnp.int32),
            pltpu.VMEM((CHUNK, LANES), jnp.float32),
            pltpu.VMEM_SHARED((np_, LANES), jnp.float32),
        ],
        compiler_params=_SC_PARAMS,
    )
    def deg_kernel(dst_hbm, ones_hbm, zeros_hbm, out_hbm, idx_v, ones_v, acc_s):
        cid = lax.axis_index("c")
        sid = lax.axis_index("s")
        w = cid * NS + sid
        base = sid * rpt
        pltpu.sync_copy(zeros_hbm, acc_s.at[pl.ds(base, rpt)])
        pltpu.sync_copy(ones_hbm, ones_v)
        pltpu.sync_copy(dst_hbm.at[w], idx_v)
        plsc.subcore_barrier()

        @pl.loop(0, chunks)
        def _(c):
            pltpu.sync_copy(ones_v, acc_s.at[idx_v.at[c]], add=True)

        plsc.subcore_barrier()
        pltpu.sync_copy(acc_s.at[pl.ds(base, rpt)],
                        out_hbm.at[cid, pl.ds(base, rpt)])

    return deg_kernel(dstp, ones_rows, zeros_rows)


def _sc_edge_pass(table, srcp, dstp, np_, h, chunks):
    rpt = np_ // NS

    @functools.partial(
        pl.kernel,
        out_type=jax.ShapeDtypeStruct((NC, np_, h), jnp.float32),
        mesh=_sc_mesh(),
        scratch_types=[
            pltpu.VMEM((chunks, CHUNK), jnp.int32),
            pltpu.VMEM((chunks, CHUNK), jnp.int32),
            pltpu.VMEM((CHUNK, h), jnp.float32),
            pltpu.VMEM_SHARED((np_, h), jnp.float32),
            pltpu.SemaphoreType.DMA,
        ],
        compiler_params=_SC_PARAMS,
    )
    def edge_kernel(table_hbm, src_hbm, dst_hbm, out_hbm,
                    src_v, dst_v, buf_v, acc_s, sem):
        cid = lax.axis_index("c")
        sid = lax.axis_index("s")
        w = cid * NS + sid
        base = sid * rpt
        pltpu.sync_copy(table_hbm.at[pl.ds(base, rpt)],
                        acc_s.at[pl.ds(base, rpt)])
        pltpu.sync_copy(src_hbm.at[w], src_v)
        pltpu.sync_copy(dst_hbm.at[w], dst_v)
        plsc.subcore_barrier()

        @pl.loop(0, chunks)
        def _(c):
            pltpu.async_copy(table_hbm.at[src_v.at[c]], buf_v, sem).wait()
            pltpu.sync_copy(buf_v, acc_s.at[dst_v.at[c]], add=True)

        plsc.subcore_barrier()
        pltpu.sync_copy(acc_s.at[pl.ds(base, rpt)],
                        out_hbm.at[cid, pl.ds(base, rpt)])

    return edge_kernel(table, srcp, dstp)


def _tc_pre(xp, W0, b0, W1):
    def body(x_ref, w0_ref, b0_ref, w1_ref, o_ref):
        hh = _MM(x_ref[...], w0_ref[...]) + b0_ref[...][None, :]
        o_ref[...] = _MM(hh, w1_ref[...])

    out = jax.ShapeDtypeStruct((xp.shape[0], W1.shape[1]), jnp.float32)
    return pl.pallas_call(body, out_shape=out)(xp, W0, b0, W1)


def _tc_scale(degp, t1):
    np_, h = t1.shape

    def body(deg_ref, t_ref, dinv_ref, tab_ref):
        deg = 1.0 + deg_ref[0] + deg_ref[1]
        dinv = lax.rsqrt(deg)[:, 0:1]
        dinv64 = jnp.broadcast_to(dinv, (np_, h))
        dinv_ref[...] = dinv64
        tab_ref[...] = t_ref[...] * dinv64

    outs = (jax.ShapeDtypeStruct((np_, h), jnp.float32),
            jax.ShapeDtypeStruct((np_, h), jnp.float32))
    return pl.pallas_call(body, out_shape=outs)(degp, t1)


def _tc_layer(p, table_prev, dinv64, b_prev, W_next):
    np_, h = table_prev.shape

    def body(p_ref, tab_ref, dinv_ref, b_ref, w_ref, o_ref):
        agg = p_ref[0] + p_ref[1] - tab_ref[...]
        hcur = jnp.maximum(dinv_ref[...] * agg + b_ref[...][None, :], 0.0)
        o_ref[...] = _MM(hcur, w_ref[...]) * dinv_ref[...]

    out = jax.ShapeDtypeStruct((np_, h), jnp.float32)
    return pl.pallas_call(body, out_shape=out)(p, table_prev, dinv64, b_prev,
                                               W_next)


def _tc_final(p, table3, dinv64, b3, n_real, ps,
              Wp1, bp1, Wp2, bp2, Wph1, bph1, Wph2, bph2, Wvh1, bvh1,
              Wvh2, bvh2):
    np_, h = table3.shape

    def body(p_ref, tab_ref, dinv_ref, b_ref, ps_ref,
             wp1_ref, bp1_ref, wp2_ref, bp2_ref,
             wph1_ref, bph1_ref, wph2_ref, bph2_ref,
             wvh1_ref, bvh1_ref, wvh2_ref, bvh2_ref,
             pol_ref, val_ref):
        r16 = lambda v: v.astype(jnp.bfloat16).astype(jnp.float32)

        agg = p_ref[0] + p_ref[1] - tab_ref[...]
        hcur = jnp.maximum(dinv_ref[...] * agg + b_ref[...][None, :], 0.0)
        row = lax.broadcasted_iota(jnp.int32, (np_, h), 0)
        hmask = jnp.where(row < n_real, hcur, 0.0)
        ge = r16(jnp.sum(hmask, axis=0, keepdims=True)
                 * jnp.float32(1.0 / n_real))

        p1h = jnp.maximum(_MM(ps_ref[...], wp1_ref[...])
                          + bp1_ref[...][None, :], 0.0)
        pe = r16(_MM(r16(p1h), wp2_ref[...]) + bp2_ref[...][None, :])

        ph = jnp.maximum(_MM(ge, wph1_ref[pl.ds(0, h), :])
                         + _MM(pe, wph1_ref[pl.ds(h, h), :])
                         + bph1_ref[...][None, :], 0.0)
        pol_ref[...] = _MM(r16(ph), wph2_ref[...]) + bph2_ref[...][None, :]

        vh = jnp.maximum(_MM(ge, wvh1_ref[pl.ds(0, h), :])
                         + _MM(pe, wvh1_ref[pl.ds(h, h), :])
                         + bvh1_ref[...][None, :], 0.0)
        val_ref[...] = _MMX(vh, wvh2_ref[...]) + bvh2_ref[...][None, :]

    outs = (jax.ShapeDtypeStruct((1, Wph2.shape[1]), jnp.float32),
            jax.ShapeDtypeStruct((1, Wvh2.shape[1]), jnp.float32))
    return pl.pallas_call(body, out_shape=outs)(
        p, table3, dinv64, b3, ps, Wp1, bp1, Wp2, bp2,
        Wph1, bph1, Wph2, bph2, Wvh1, bvh1, Wvh2, bvh2)


def kernel(x, edge_index, private_state, W0, b0, W1, b1, W2, b2, W3, b3,
           Wp1, bp1, Wp2, bp2, Wph1, bph1, Wph2, bph2, Wvh1, bvh1, Wvh2, bvh2):
    n, _ = x.shape
    h = W1.shape[0]
    e = edge_index.shape[1]

    np_ = _round_up(n + 1, NS * 8)
    chunks = _round_up(-(-e // NW), CHUNK) // CHUNK
    ept = chunks * CHUNK
    pad = NW * ept - e

    src = edge_index[0]
    dst = edge_index[1]
    srcp = jnp.concatenate(
        [src, jnp.zeros((pad,), jnp.int32)]).reshape(NW, chunks, CHUNK)
    dstp = jnp.concatenate(
        [dst, jnp.full((pad,), n, jnp.int32)]).reshape(NW, chunks, CHUNK)

    xp = jnp.pad(x, ((0, np_ - n), (0, 0)))
    ones_rows = jnp.ones((CHUNK, LANES), jnp.float32)
    zeros_rows = jnp.zeros((np_ // NS, LANES), jnp.float32)

    degp = _sc_degree(dstp, ones_rows, zeros_rows, np_, chunks)
    t1 = _tc_pre(xp, W0, b0, W1)
    dinv64, table1 = _tc_scale(degp, t1)

    p1 = _sc_edge_pass(table1, srcp, dstp, np_, h, chunks)
    table2 = _tc_layer(p1, table1, dinv64, b1, W2)
    p2 = _sc_edge_pass(table2, srcp, dstp, np_, h, chunks)
    table3 = _tc_layer(p2, table2, dinv64, b2, W3)
    p3 = _sc_edge_pass(table3, srcp, dstp, np_, h, chunks)

    return _tc_final(p3, table3, dinv64, b3, n, private_state,
                     Wp1, bp1, Wp2, bp2, Wph1, bph1, Wph2, bph2,
                     Wvh1, bvh1, Wvh2, bvh2)


# speedup vs baseline: 34.8475x; 2.1440x over previous
import functools

import jax
import jax.numpy as jnp
from jax import lax
from jax.experimental import pallas as pl
from jax.experimental.pallas import tpu as pltpu
from jax.experimental.pallas import tpu_sc as plsc

NC = 2
NS = 16
NW = NC * NS
LANES = 16
CHUNK = 128

_MM = functools.partial(jnp.dot, preferred_element_type=jnp.float32)
_MMX = functools.partial(jnp.dot, preferred_element_type=jnp.float32,
                         precision=lax.Precision.HIGHEST)


def _round_up(a, b):
    return (a + b - 1) // b * b


def _sc_mesh():
    return plsc.VectorSubcoreMesh(core_axis_name="c", subcore_axis_name="s",
                                  num_cores=NC, num_subcores=NS)


_SC_PARAMS = pltpu.CompilerParams(use_tc_tiling_on_sc=False)


def _sc_degree(dstp, ones_rows, zeros_rows, np_, chunks):
    rpt = np_ // NS

    @functools.partial(
        pl.kernel,
        out_type=jax.ShapeDtypeStruct((NC, np_, LANES), jnp.float32),
        mesh=_sc_mesh(),
        scratch_types=[
            pltpu.VMEM((chunks, CHUNK), jnp.int32),
            pltpu.VMEM((CHUNK, LANES), jnp.float32),
            pltpu.VMEM_SHARED((np_, LANES), jnp.float32),
        ],
        compiler_params=_SC_PARAMS,
    )
    def deg_kernel(dst_hbm, ones_hbm, zeros_hbm, out_hbm, idx_v, ones_v, acc_s):
        cid = lax.axis_index("c")
        sid = lax.axis_index("s")
        w = cid * NS + sid
        base = sid * rpt
        pltpu.sync_copy(zeros_hbm, acc_s.at[pl.ds(base, rpt)])
        pltpu.sync_copy(ones_hbm, ones_v)
        pltpu.sync_copy(dst_hbm.at[w], idx_v)
        plsc.subcore_barrier()

        @pl.loop(0, chunks)
        def _(c):
            pltpu.sync_copy(ones_v, acc_s.at[idx_v.at[c]], add=True)

        plsc.subcore_barrier()
        pltpu.sync_copy(acc_s.at[pl.ds(base, rpt)],
                        out_hbm.at[cid, pl.ds(base, rpt)])

    return deg_kernel(dstp, ones_rows, zeros_rows)


def _sc_edge_pass(table, srcp, dstp, np_, h, chunks):
    rpt = np_ // NS

    @functools.partial(
        pl.kernel,
        out_type=jax.ShapeDtypeStruct((NC, np_, h), jnp.float32),
        mesh=_sc_mesh(),
        scratch_types=[
            pltpu.VMEM((chunks, CHUNK), jnp.int32),
            pltpu.VMEM((chunks, CHUNK), jnp.int32),
            pltpu.VMEM((CHUNK, h), jnp.float32),
            pltpu.VMEM((CHUNK, h), jnp.float32),
            pltpu.VMEM_SHARED((np_, h), jnp.float32),
            pltpu.SemaphoreType.DMA,
            pltpu.SemaphoreType.DMA,
        ],
        compiler_params=_SC_PARAMS,
    )
    def edge_kernel(table_hbm, src_hbm, dst_hbm, out_hbm,
                    src_v, dst_v, buf0_v, buf1_v, acc_s, sem0, sem1):
        cid = lax.axis_index("c")
        sid = lax.axis_index("s")
        w = cid * NS + sid
        base = sid * rpt
        pltpu.sync_copy(table_hbm.at[pl.ds(base, rpt)],
                        acc_s.at[pl.ds(base, rpt)])
        pltpu.sync_copy(src_hbm.at[w], src_v)
        pltpu.sync_copy(dst_hbm.at[w], dst_v)
        plsc.subcore_barrier()

        pltpu.async_copy(table_hbm.at[src_v.at[0]], buf0_v, sem0)
        pltpu.async_copy(table_hbm.at[src_v.at[1]], buf1_v, sem1)

        @pl.loop(0, chunks, step=2)
        def _(c):
            pltpu.make_async_copy(table_hbm.at[src_v.at[c]], buf0_v,
                                  sem0).wait()
            pltpu.sync_copy(buf0_v, acc_s.at[dst_v.at[c]], add=True)

            @pl.when(c + 2 < chunks)
            def _():
                pltpu.async_copy(table_hbm.at[src_v.at[c + 2]], buf0_v, sem0)

            pltpu.make_async_copy(table_hbm.at[src_v.at[c + 1]], buf1_v,
                                  sem1).wait()
            pltpu.sync_copy(buf1_v, acc_s.at[dst_v.at[c + 1]], add=True)

            @pl.when(c + 3 < chunks)
            def _():
                pltpu.async_copy(table_hbm.at[src_v.at[c + 3]], buf1_v, sem1)

        plsc.subcore_barrier()
        pltpu.sync_copy(acc_s.at[pl.ds(base, rpt)],
                        out_hbm.at[cid, pl.ds(base, rpt)])

    return edge_kernel(table, srcp, dstp)


def _tc_pre(xp, W0, b0, W1):
    def body(x_ref, w0_ref, b0_ref, w1_ref, o_ref):
        hh = _MM(x_ref[...], w0_ref[...]) + b0_ref[...][None, :]
        o_ref[...] = _MM(hh, w1_ref[...])

    out = jax.ShapeDtypeStruct((xp.shape[0], W1.shape[1]), jnp.float32)
    return pl.pallas_call(body, out_shape=out)(xp, W0, b0, W1)


def _tc_scale(degp, t1):
    np_, h = t1.shape

    def body(deg_ref, t_ref, dinv_ref, tab_ref):
        deg = 1.0 + deg_ref[0] + deg_ref[1]
        dinv = lax.rsqrt(deg)[:, 0:1]
        dinv64 = jnp.broadcast_to(dinv, (np_, h))
        dinv_ref[...] = dinv64
        tab_ref[...] = t_ref[...] * dinv64

    outs = (jax.ShapeDtypeStruct((np_, h), jnp.float32),
            jax.ShapeDtypeStruct((np_, h), jnp.float32))
    return pl.pallas_call(body, out_shape=outs)(degp, t1)


def _tc_layer(p, table_prev, dinv64, b_prev, W_next):
    np_, h = table_prev.shape

    def body(p_ref, tab_ref, dinv_ref, b_ref, w_ref, o_ref):
        agg = p_ref[0] + p_ref[1] - tab_ref[...]
        hcur = jnp.maximum(dinv_ref[...] * agg + b_ref[...][None, :], 0.0)
        o_ref[...] = _MM(hcur, w_ref[...]) * dinv_ref[...]

    out = jax.ShapeDtypeStruct((np_, h), jnp.float32)
    return pl.pallas_call(body, out_shape=out)(p, table_prev, dinv64, b_prev,
                                               W_next)


def _tc_final(p, table3, dinv64, b3, n_real, ps,
              Wp1, bp1, Wp2, bp2, Wph1, bph1, Wph2, bph2, Wvh1, bvh1,
              Wvh2, bvh2):
    np_, h = table3.shape

    def body(p_ref, tab_ref, dinv_ref, b_ref, ps_ref,
             wp1_ref, bp1_ref, wp2_ref, bp2_ref,
             wph1_ref, bph1_ref, wph2_ref, bph2_ref,
             wvh1_ref, bvh1_ref, wvh2_ref, bvh2_ref,
             pol_ref, val_ref):
        r16 = lambda v: v.astype(jnp.bfloat16).astype(jnp.float32)

        agg = p_ref[0] + p_ref[1] - tab_ref[...]
        hcur = jnp.maximum(dinv_ref[...] * agg + b_ref[...][None, :], 0.0)
        row = lax.broadcasted_iota(jnp.int32, (np_, h), 0)
        hmask = jnp.where(row < n_real, hcur, 0.0)
        ge = r16(jnp.sum(hmask, axis=0, keepdims=True)
                 * jnp.float32(1.0 / n_real))

        p1h = jnp.maximum(_MM(ps_ref[...], wp1_ref[...])
                          + bp1_ref[...][None, :], 0.0)
        pe = r16(_MM(r16(p1h), wp2_ref[...]) + bp2_ref[...][None, :])

        ph = jnp.maximum(_MM(ge, wph1_ref[pl.ds(0, h), :])
                         + _MM(pe, wph1_ref[pl.ds(h, h), :])
                         + bph1_ref[...][None, :], 0.0)
        pol_ref[...] = _MM(r16(ph), wph2_ref[...]) + bph2_ref[...][None, :]

        vh = jnp.maximum(_MM(ge, wvh1_ref[pl.ds(0, h), :])
                         + _MM(pe, wvh1_ref[pl.ds(h, h), :])
                         + bvh1_ref[...][None, :], 0.0)
        val_ref[...] = _MMX(vh, wvh2_ref[...]) + bvh2_ref[...][None, :]

    outs = (jax.ShapeDtypeStruct((1, Wph2.shape[1]), jnp.float32),
            jax.ShapeDtypeStruct((1, Wvh2.shape[1]), jnp.float32))
    return pl.pallas_call(body, out_shape=outs)(
        p, table3, dinv64, b3, ps, Wp1, bp1, Wp2, bp2,
        Wph1, bph1, Wph2, bph2, Wvh1, bvh1, Wvh2, bvh2)


def kernel(x, edge_index, private_state, W0, b0, W1, b1, W2, b2, W3, b3,
           Wp1, bp1, Wp2, bp2, Wph1, bph1, Wph2, bph2, Wvh1, bvh1, Wvh2, bvh2):
    n, _ = x.shape
    h = W1.shape[0]
    e = edge_index.shape[1]

    np_ = _round_up(n + 1, NS * 8)
    chunks = 2 * (_round_up(-(-e // NW), 2 * CHUNK) // (2 * CHUNK))
    ept = chunks * CHUNK
    pad = NW * ept - e

    src = edge_index[0]
    dst = edge_index[1]
    pad_i = jnp.arange(pad, dtype=jnp.int32)
    srcp = jnp.concatenate([src, pad_i % n]).reshape(NW, chunks, CHUNK)
    dstp = jnp.concatenate(
        [dst, n + pad_i % (np_ - n)]).reshape(NW, chunks, CHUNK)

    xp = jnp.pad(x, ((0, np_ - n), (0, 0)))
    ones_rows = jnp.ones((CHUNK, LANES), jnp.float32)
    zeros_rows = jnp.zeros((np_ // NS, LANES), jnp.float32)

    degp = _sc_degree(dstp, ones_rows, zeros_rows, np_, chunks)
    t1 = _tc_pre(xp, W0, b0, W1)
    dinv64, table1 = _tc_scale(degp, t1)

    p1 = _sc_edge_pass(table1, srcp, dstp, np_, h, chunks)
    table2 = _tc_layer(p1, table1, dinv64, b1, W2)
    p2 = _sc_edge_pass(table2, srcp, dstp, np_, h, chunks)
    table3 = _tc_layer(p2, table2, dinv64, b2, W3)
    p3 = _sc_edge_pass(table3, srcp, dstp, np_, h, chunks)

    return _tc_final(p3, table3, dinv64, b3, n, private_state,
                     Wp1, bp1, Wp2, bp2, Wph1, bph1, Wph2, bph2,
                     Wvh1, bvh1, Wvh2, bvh2)
